# baseline (speedup 1.0000x reference)
import functools

import jax
import jax.numpy as jnp
from jax.experimental import pallas as pl
from jax.experimental.pallas import tpu as pltpu


def _accum_block(first, e, r, rl_col, lab_row, q2_row, acc_ref):
    blk, _ = r.shape
    b = e.shape[0]
    acc = jax.lax.dot_general(
        r, e, (((1,), (1,)), ((), ())),
        preferred_element_type=jnp.float32,
        precision=jax.lax.Precision.HIGHEST,
    )
    m2 = jnp.sum(r * r, axis=1, keepdims=True)
    d2 = (m2 + q2_row) - 2.0 * acc
    dist = jnp.sqrt(jnp.maximum(d2, 0.0) + 1e-12)

    same = rl_col == lab_row
    if first:
        row_i = jax.lax.broadcasted_iota(jnp.int32, (blk, b), 0)
        col_i = jax.lax.broadcasted_iota(jnp.int32, (blk, b), 1)
        posm = same & (row_i != col_i)
    else:
        posm = same

    pos_s = jnp.sum(jnp.where(posm, dist, 0.0), axis=0, keepdims=True)
    pos_c = jnp.sum(posm.astype(jnp.float32), axis=0, keepdims=True)
    nl = jnp.where(same, 0.0, jnp.maximum(1.0 - dist, 0.0))
    neg_s = jnp.sum(nl, axis=0, keepdims=True)
    neg_c = jnp.sum((nl > 0.0).astype(jnp.float32), axis=0, keepdims=True)

    acc_ref[...] += jnp.concatenate([pos_s, pos_c, neg_s, neg_c], axis=0)


def _body(e_ref, lab_row_ref, lab_col_ref, mem_ref, mlab_ref, out_ref,
          acc_ref, q2_ref):
    j = pl.program_id(0)
    e = e_ref[...]
    lab_row = lab_row_ref[...]

    @pl.when(j == 0)
    def _first():
        acc_ref[...] = jnp.zeros_like(acc_ref)
        esq = e * e
        q2 = jax.lax.dot_general(
            jnp.ones((8, e.shape[1]), jnp.float32), esq,
            (((1,), (1,)), ((), ())),
            preferred_element_type=jnp.float32,
            precision=jax.lax.Precision.HIGHEST,
        )[0:1]
        q2_ref[...] = q2
        _accum_block(True, e, e, lab_col_ref[...], lab_row, q2, acc_ref)

    @pl.when(j > 0)
    def _rest():
        _accum_block(False, e, mem_ref[...], mlab_ref[...], lab_row,
                     q2_ref[...], acc_ref)

    @pl.when(j == pl.num_programs(0) - 1)
    def _final():
        s = jnp.sum(acc_ref[...], axis=1, keepdims=True)
        num = jnp.concatenate([s[0:1], s[2:3]], axis=0)
        den = jnp.maximum(jnp.concatenate([s[1:2], s[3:4]], axis=0), 1.0)
        out_ref[...] = jnp.sum(num / den, axis=0, keepdims=True)


def kernel(embeddings, labels, memory_emb, memory_labels):
    b, d = embeddings.shape
    m = memory_emb.shape[0]
    blk = b
    grid = m // blk

    lab_row = labels.reshape(1, b)
    lab_col = labels.reshape(b, 1)
    mlab_col = memory_labels.reshape(m, 1)

    out = pl.pallas_call(
        _body,
        grid=(grid,),
        in_specs=[
            pl.BlockSpec((b, d), lambda j: (0, 0)),
            pl.BlockSpec((1, b), lambda j: (0, 0)),
            pl.BlockSpec((b, 1), lambda j: (0, 0)),
            pl.BlockSpec((blk, d), lambda j: (j, 0)),
            pl.BlockSpec((blk, 1), lambda j: (j, 0)),
        ],
        out_specs=pl.BlockSpec((1, 1), lambda j: (0, 0)),
        out_shape=jax.ShapeDtypeStruct((1, 1), jnp.float32),
        scratch_shapes=[
            pltpu.VMEM((4, b), jnp.float32),
            pltpu.VMEM((1, b), jnp.float32),
        ],
        compiler_params=pltpu.CompilerParams(
            dimension_semantics=("arbitrary",),
        ),
    )(embeddings, lab_row, lab_col, memory_emb, mlab_col)
    return out[0, 0]


# speedup vs baseline: 1.7270x; 1.7270x over previous
import functools

import jax
import jax.numpy as jnp
from jax.experimental import pallas as pl
from jax.experimental.pallas import tpu as pltpu


def _accum_block(first, eaug, r, rl_col, lab_row, acc_ref):
    blk = r.shape[0]
    b = eaug.shape[0]
    m2 = jnp.sum(r * r, axis=1, keepdims=True)
    raug = jnp.concatenate(
        [r, m2, jnp.ones((blk, 1), jnp.float32)], axis=1)
    d2 = jax.lax.dot_general(
        raug, eaug, (((1,), (1,)), ((), ())),
        preferred_element_type=jnp.float32,
    )
    dist = jnp.sqrt(jnp.maximum(d2, 0.0) + 1e-12)

    same = rl_col == lab_row
    if first:
        row_i = jax.lax.broadcasted_iota(jnp.int32, (blk, b), 0)
        col_i = jax.lax.broadcasted_iota(jnp.int32, (blk, b), 1)
        posm = same & (row_i != col_i)
    else:
        posm = same

    posv = jnp.where(posm, dist, 0.0)
    posc = jnp.where(posm, 1.0, 0.0)
    negv = jnp.where(same, 0.0, jnp.maximum(1.0 - dist, 0.0))
    negc = jnp.where(negv > 0.0, 1.0, 0.0)

    pos_s = jnp.sum(posv, axis=0, keepdims=True)
    pos_c = jnp.sum(posc, axis=0, keepdims=True)
    neg_s = jnp.sum(negv, axis=0, keepdims=True)
    neg_c = jnp.sum(negc, axis=0, keepdims=True)

    acc_ref[...] += jnp.concatenate([pos_s, pos_c, neg_s, neg_c], axis=0)


def _body(e_ref, lab_row_ref, lab_col_ref, mem_ref, mlab_ref, out_ref,
          acc_ref, eaug_ref):
    j = pl.program_id(0)
    lab_row = lab_row_ref[...]

    @pl.when(j == 0)
    def _first():
        acc_ref[...] = jnp.zeros_like(acc_ref)
        e = e_ref[...]
        b = e.shape[0]
        q2 = jnp.sum(e * e, axis=1, keepdims=True)
        eaug = jnp.concatenate(
            [-2.0 * e, jnp.ones((b, 1), jnp.float32), q2], axis=1)
        eaug_ref[...] = eaug
        _accum_block(True, eaug, e, lab_col_ref[...], lab_row, acc_ref)

    @pl.when(j > 0)
    def _rest():
        _accum_block(False, eaug_ref[...], mem_ref[...], mlab_ref[...],
                     lab_row, acc_ref)

    @pl.when(j == pl.num_programs(0) - 1)
    def _final():
        s = jnp.sum(acc_ref[...], axis=1, keepdims=True)
        num = jnp.concatenate([s[0:1], s[2:3]], axis=0)
        den = jnp.maximum(jnp.concatenate([s[1:2], s[3:4]], axis=0), 1.0)
        out_ref[...] = jnp.sum(num / den, axis=0, keepdims=True)


def kernel(embeddings, labels, memory_emb, memory_labels):
    b, d = embeddings.shape
    m = memory_emb.shape[0]
    blk = b
    grid = m // blk

    lab_row = labels.reshape(1, b)
    lab_col = labels.reshape(b, 1)
    mlab_col = memory_labels.reshape(m, 1)

    out = pl.pallas_call(
        _body,
        grid=(grid,),
        in_specs=[
            pl.BlockSpec((b, d), lambda j: (0, 0)),
            pl.BlockSpec((1, b), lambda j: (0, 0)),
            pl.BlockSpec((b, 1), lambda j: (0, 0)),
            pl.BlockSpec((blk, d), lambda j: (j, 0)),
            pl.BlockSpec((blk, 1), lambda j: (j, 0)),
        ],
        out_specs=pl.BlockSpec((1, 1), lambda j: (0, 0)),
        out_shape=jax.ShapeDtypeStruct((1, 1), jnp.float32),
        scratch_shapes=[
            pltpu.VMEM((4, b), jnp.float32),
            pltpu.VMEM((b, d + 2), jnp.float32),
        ],
        compiler_params=pltpu.CompilerParams(
            dimension_semantics=("arbitrary",),
        ),
    )(embeddings, lab_row, lab_col, memory_emb, mlab_col)
    return out[0, 0]


# speedup vs baseline: 2.1673x; 1.2550x over previous
import functools

import jax
import jax.numpy as jnp
from jax.experimental import pallas as pl
from jax.experimental.pallas import tpu as pltpu


def _accum_block(first, eaug, r, rl_col, lab_row, acc_ref):
    blk = r.shape[0]
    b = eaug.shape[0]
    m2 = jnp.sum(r * r, axis=1, keepdims=True)
    raug = jnp.concatenate(
        [r, m2, jnp.ones((blk, 1), jnp.float32)], axis=1)
    d2 = jax.lax.dot_general(
        raug, eaug, (((1,), (1,)), ((), ())),
        preferred_element_type=jnp.float32,
    )
    d2c = jnp.maximum(d2, 1e-12)
    dist = d2c * jax.lax.rsqrt(d2c)

    same = rl_col == lab_row
    if first:
        row_i = jax.lax.broadcasted_iota(jnp.int32, (blk, b), 0)
        col_i = jax.lax.broadcasted_iota(jnp.int32, (blk, b), 1)
        posm = same & (row_i != col_i)
    else:
        posm = same

    posv = jnp.where(posm, dist, 0.0)
    posc = jnp.where(posm, 1.0, 0.0)
    negv = jnp.where(same, 0.0, jnp.maximum(1.0 - dist, 0.0))
    negc = jnp.where(negv > 0.0, 1.0, 0.0)

    pos_s = jnp.sum(posv, axis=0, keepdims=True)
    pos_c = jnp.sum(posc, axis=0, keepdims=True)
    neg_s = jnp.sum(negv, axis=0, keepdims=True)
    neg_c = jnp.sum(negc, axis=0, keepdims=True)

    acc_ref[...] += jnp.concatenate([pos_s, pos_c, neg_s, neg_c], axis=0)


def _body(e_ref, lab_row_ref, lab_col_ref, mem_ref, mlab_ref, out_ref,
          acc_ref, eaug_ref):
    j = pl.program_id(0)
    lab_row = lab_row_ref[...]

    @pl.when(j == 0)
    def _first():
        acc_ref[...] = jnp.zeros_like(acc_ref)
        e = e_ref[...]
        b = e.shape[0]
        q2 = jnp.sum(e * e, axis=1, keepdims=True)
        eaug = jnp.concatenate(
            [-2.0 * e, jnp.ones((b, 1), jnp.float32), q2], axis=1)
        eaug_ref[...] = eaug
        _accum_block(True, eaug, e, lab_col_ref[...], lab_row, acc_ref)

    @pl.when(j > 0)
    def _rest():
        _accum_block(False, eaug_ref[...], mem_ref[...], mlab_ref[...],
                     lab_row, acc_ref)

    @pl.when(j == pl.num_programs(0) - 1)
    def _final():
        s = jnp.sum(acc_ref[...], axis=1, keepdims=True)
        num = jnp.concatenate([s[0:1], s[2:3]], axis=0)
        den = jnp.maximum(jnp.concatenate([s[1:2], s[3:4]], axis=0), 1.0)
        out_ref[...] = jnp.sum(num / den, axis=0, keepdims=True)


def kernel(embeddings, labels, memory_emb, memory_labels):
    b, d = embeddings.shape
    m = memory_emb.shape[0]
    blk = b
    grid = m // blk

    lab_row = labels.reshape(1, b)
    lab_col = labels.reshape(b, 1)
    mlab_col = memory_labels.reshape(m, 1)

    out = pl.pallas_call(
        _body,
        grid=(grid,),
        in_specs=[
            pl.BlockSpec((b, d), lambda j: (0, 0)),
            pl.BlockSpec((1, b), lambda j: (0, 0)),
            pl.BlockSpec((b, 1), lambda j: (0, 0)),
            pl.BlockSpec((blk, d), lambda j: (j, 0)),
            pl.BlockSpec((blk, 1), lambda j: (j, 0)),
        ],
        out_specs=pl.BlockSpec((1, 1), lambda j: (0, 0)),
        out_shape=jax.ShapeDtypeStruct((1, 1), jnp.float32),
        scratch_shapes=[
            pltpu.VMEM((4, b), jnp.float32),
            pltpu.VMEM((b, d + 2), jnp.float32),
        ],
        compiler_params=pltpu.CompilerParams(
            dimension_semantics=("arbitrary",),
        ),
    )(embeddings, lab_row, lab_col, memory_emb, mlab_col)
    return out[0, 0]


# speedup vs baseline: 2.3116x; 1.0666x over previous
import functools

import jax
import jax.numpy as jnp
from jax.experimental import pallas as pl
from jax.experimental.pallas import tpu as pltpu


def _accum_block(first, eaug, r, lab_col, rl_row, acc_ref):
    blk = r.shape[0]
    b = eaug.shape[0]
    m2 = jnp.sum(r * r, axis=1, keepdims=True)
    raug = jnp.concatenate(
        [r, m2, jnp.ones((blk, 1), jnp.float32)], axis=1)
    d2 = jax.lax.dot_general(
        eaug, raug, (((1,), (1,)), ((), ())),
        preferred_element_type=jnp.float32,
    )
    d2c = jnp.maximum(d2, 1e-12)
    dist = d2c * jax.lax.rsqrt(d2c)

    same = lab_col == rl_row
    if first:
        row_i = jax.lax.broadcasted_iota(jnp.int32, (b, blk), 0)
        col_i = jax.lax.broadcasted_iota(jnp.int32, (b, blk), 1)
        posm = same & (row_i != col_i)
    else:
        posm = same

    posv = jnp.where(posm, dist, 0.0)
    posc = jnp.where(posm, 1.0, 0.0)
    negv = jnp.where(same, 0.0, jnp.maximum(1.0 - dist, 0.0))
    negc = jnp.where(negv > 0.0, 1.0, 0.0)

    pos_s = jnp.sum(posv, axis=0, keepdims=True)
    pos_c = jnp.sum(posc, axis=0, keepdims=True)
    neg_s = jnp.sum(negv, axis=0, keepdims=True)
    neg_c = jnp.sum(negc, axis=0, keepdims=True)

    acc_ref[...] += jnp.concatenate([pos_s, pos_c, neg_s, neg_c], axis=0)


def _body(e_ref, lab_row_ref, lab_col_ref, mem_ref, mlab_ref, out_ref,
          acc_ref, eaug_ref):
    j = pl.program_id(0)
    lab_col = lab_col_ref[...]

    @pl.when(j == 0)
    def _first():
        acc_ref[...] = jnp.zeros_like(acc_ref)
        e = e_ref[...]
        b = e.shape[0]
        q2 = jnp.sum(e * e, axis=1, keepdims=True)
        eaug = jnp.concatenate(
            [-2.0 * e, jnp.ones((b, 1), jnp.float32), q2], axis=1)
        eaug_ref[...] = eaug
        _accum_block(True, eaug, e, lab_col, lab_row_ref[...], acc_ref)

    @pl.when(j > 0)
    def _rest():
        _accum_block(False, eaug_ref[...], mem_ref[...], lab_col,
                     mlab_ref[0], acc_ref)

    @pl.when(j == pl.num_programs(0) - 1)
    def _final():
        s = jnp.sum(acc_ref[...], axis=1, keepdims=True)
        num = jnp.concatenate([s[0:1], s[2:3]], axis=0)
        den = jnp.maximum(jnp.concatenate([s[1:2], s[3:4]], axis=0), 1.0)
        out_ref[...] = jnp.sum(num / den, axis=0, keepdims=True)


def kernel(embeddings, labels, memory_emb, memory_labels):
    b, d = embeddings.shape
    m = memory_emb.shape[0]
    blk = b
    grid = m // blk

    lab_row = labels.reshape(1, b)
    lab_col = labels.reshape(b, 1)
    mlab = memory_labels.reshape(grid, 1, blk)

    out = pl.pallas_call(
        _body,
        grid=(grid,),
        in_specs=[
            pl.BlockSpec((b, d), lambda j: (0, 0)),
            pl.BlockSpec((1, b), lambda j: (0, 0)),
            pl.BlockSpec((b, 1), lambda j: (0, 0)),
            pl.BlockSpec((blk, d), lambda j: (j, 0)),
            pl.BlockSpec((1, 1, blk), lambda j: (j, 0, 0)),
        ],
        out_specs=pl.BlockSpec((1, 1), lambda j: (0, 0)),
        out_shape=jax.ShapeDtypeStruct((1, 1), jnp.float32),
        scratch_shapes=[
            pltpu.VMEM((4, blk), jnp.float32),
            pltpu.VMEM((b, d + 2), jnp.float32),
        ],
        compiler_params=pltpu.CompilerParams(
            dimension_semantics=("arbitrary",),
        ),
    )(embeddings, lab_row, lab_col, memory_emb, mlab)
    return out[0, 0]


# speedup vs baseline: 2.5362x; 1.0972x over previous
import functools

import jax
import jax.numpy as jnp
from jax.experimental import pallas as pl
from jax.experimental.pallas import tpu as pltpu


def _accum_block(first, eaug, r, lab_col, rl_row, acc_ref):
    blk = r.shape[0]
    b = eaug.shape[0]
    m2 = jnp.sum(r * r, axis=1, keepdims=True)
    raug = jnp.concatenate(
        [r, m2, jnp.ones((blk, 1), jnp.float32)], axis=1)
    d2 = jax.lax.dot_general(
        eaug, raug, (((1,), (1,)), ((), ())),
        preferred_element_type=jnp.float32,
    )
    d2c = jnp.maximum(d2, 1e-12)
    dist = d2c * jax.lax.rsqrt(d2c)

    same = lab_col == rl_row
    if first:
        row_i = jax.lax.broadcasted_iota(jnp.int32, (b, blk), 0)
        col_i = jax.lax.broadcasted_iota(jnp.int32, (b, blk), 1)
        posm = same & (row_i != col_i)
    else:
        posm = same

    posv = jnp.where(posm, dist, 0.0)
    posc = jnp.where(posm, 1.0, 0.0)
    negv = jnp.where(same, 0.0, jnp.maximum(1.0 - dist, 0.0))
    negc = jnp.ceil(negv)

    pos_s = jnp.sum(posv, axis=0, keepdims=True)
    pos_c = jnp.sum(posc, axis=0, keepdims=True)
    neg_s = jnp.sum(negv, axis=0, keepdims=True)
    neg_c = jnp.sum(negc, axis=0, keepdims=True)

    acc_ref[...] += jnp.concatenate([pos_s, pos_c, neg_s, neg_c], axis=0)


def _body(e_ref, lab_row_ref, lab_col_ref, mem_ref, mlab_ref, out_ref,
          acc_ref, eaug_ref):
    j = pl.program_id(0)
    lab_col = lab_col_ref[...]

    @pl.when(j == 0)
    def _first():
        acc_ref[...] = jnp.zeros_like(acc_ref)
        e = e_ref[...]
        b = e.shape[0]
        q2 = jnp.sum(e * e, axis=1, keepdims=True)
        eaug = jnp.concatenate(
            [-2.0 * e, jnp.ones((b, 1), jnp.float32), q2], axis=1)
        eaug_ref[...] = eaug
        blk = mem_ref.shape[0]
        r0 = jnp.concatenate([e, mem_ref[b:blk, :]], axis=0)
        rl0 = jnp.concatenate(
            [lab_row_ref[...], mlab_ref[0][:, b:blk]], axis=1)
        _accum_block(True, eaug, r0, lab_col, rl0, acc_ref)

    @pl.when(j > 0)
    def _rest():
        _accum_block(False, eaug_ref[...], mem_ref[...], lab_col,
                     mlab_ref[0], acc_ref)

    @pl.when(j == pl.num_programs(0) - 1)
    def _final():
        s = jnp.sum(acc_ref[...], axis=1, keepdims=True)
        num = jnp.concatenate([s[0:1], s[2:3]], axis=0)
        den = jnp.maximum(jnp.concatenate([s[1:2], s[3:4]], axis=0), 1.0)
        out_ref[...] = jnp.sum(num / den, axis=0, keepdims=True)


def kernel(embeddings, labels, memory_emb, memory_labels):
    b, d = embeddings.shape
    m = memory_emb.shape[0]
    blk = 2 * b
    grid = m // blk

    lab_row = labels.reshape(1, b)
    lab_col = labels.reshape(b, 1)
    mlab = memory_labels.reshape(grid, 1, blk)

    out = pl.pallas_call(
        _body,
        grid=(grid,),
        in_specs=[
            pl.BlockSpec((b, d), lambda j: (0, 0)),
            pl.BlockSpec((1, b), lambda j: (0, 0)),
            pl.BlockSpec((b, 1), lambda j: (0, 0)),
            pl.BlockSpec((blk, d), lambda j: (j, 0)),
            pl.BlockSpec((1, 1, blk), lambda j: (j, 0, 0)),
        ],
        out_specs=pl.BlockSpec((1, 1), lambda j: (0, 0)),
        out_shape=jax.ShapeDtypeStruct((1, 1), jnp.float32),
        scratch_shapes=[
            pltpu.VMEM((4, blk), jnp.float32),
            pltpu.VMEM((b, d + 2), jnp.float32),
        ],
        compiler_params=pltpu.CompilerParams(
            dimension_semantics=("arbitrary",),
        ),
    )(embeddings, lab_row, lab_col, memory_emb, mlab)
    return out[0, 0]


# speedup vs baseline: 2.5803x; 1.0174x over previous
import functools

import jax
import jax.numpy as jnp
from jax.experimental import pallas as pl
from jax.experimental.pallas import tpu as pltpu


def _accum_block(first, eaug, r, lab_col, rl_row, acc_ref):
    blk = r.shape[0]
    b = eaug.shape[0]
    m2 = jnp.sum(r * r, axis=1, keepdims=True)
    raug = jnp.concatenate(
        [r, m2, jnp.ones((blk, 1), jnp.float32)], axis=1)
    d2 = jax.lax.dot_general(
        eaug, raug, (((1,), (1,)), ((), ())),
        preferred_element_type=jnp.float32,
    )
    d2c = jnp.maximum(d2, 1e-12)
    dist = d2c * jax.lax.rsqrt(d2c)

    same = lab_col == rl_row
    if first:
        row_i = jax.lax.broadcasted_iota(jnp.int32, (b, blk), 0)
        col_i = jax.lax.broadcasted_iota(jnp.int32, (b, blk), 1)
        posm = same & (row_i != col_i)
    else:
        posm = same

    posv = jnp.where(posm, dist, 0.0)
    posc = jnp.where(posm, 1.0, 0.0)
    negv = jnp.where(same, 0.0, jnp.maximum(1.0 - dist, 0.0))
    negc = jnp.ceil(negv)

    pos_s = jnp.sum(posv, axis=0, keepdims=True)
    pos_c = jnp.sum(posc, axis=0, keepdims=True)
    neg_s = jnp.sum(negv, axis=0, keepdims=True)
    neg_c = jnp.sum(negc, axis=0, keepdims=True)

    acc_ref[...] += jnp.concatenate([pos_s, pos_c, neg_s, neg_c], axis=0)


def _body(e_ref, lab_row_ref, lab_col_ref, mem_ref, mlab_ref, out_ref,
          acc_ref, eaug_ref):
    j = pl.program_id(0)
    lab_col = lab_col_ref[...]

    @pl.when(j == 0)
    def _first():
        acc_ref[...] = jnp.zeros_like(acc_ref)
        e = e_ref[...]
        b = e.shape[0]
        q2 = jnp.sum(e * e, axis=1, keepdims=True)
        eaug = jnp.concatenate(
            [-2.0 * e, jnp.ones((b, 1), jnp.float32), q2], axis=1)
        eaug_ref[...] = eaug
        blk = mem_ref.shape[0]
        r0 = jnp.concatenate([e, mem_ref[b:blk, :]], axis=0)
        rl0 = jnp.concatenate(
            [lab_row_ref[...], mlab_ref[0][:, b:blk]], axis=1)
        _accum_block(True, eaug, r0, lab_col, rl0, acc_ref)

    @pl.when(j > 0)
    def _rest():
        _accum_block(False, eaug_ref[...], mem_ref[...], lab_col,
                     mlab_ref[0], acc_ref)

    @pl.when(j == pl.num_programs(0) - 1)
    def _final():
        s = jnp.sum(acc_ref[...], axis=1, keepdims=True)
        num = jnp.concatenate([s[0:1], s[2:3]], axis=0)
        den = jnp.maximum(jnp.concatenate([s[1:2], s[3:4]], axis=0), 1.0)
        out_ref[...] = jnp.sum(num / den, axis=0, keepdims=True)


def kernel(embeddings, labels, memory_emb, memory_labels):
    b, d = embeddings.shape
    m = memory_emb.shape[0]
    blk = 4 * b
    grid = m // blk

    lab_row = labels.reshape(1, b)
    lab_col = labels.reshape(b, 1)
    mlab = memory_labels.reshape(grid, 1, blk)

    out = pl.pallas_call(
        _body,
        grid=(grid,),
        in_specs=[
            pl.BlockSpec((b, d), lambda j: (0, 0)),
            pl.BlockSpec((1, b), lambda j: (0, 0)),
            pl.BlockSpec((b, 1), lambda j: (0, 0)),
            pl.BlockSpec((blk, d), lambda j: (j, 0)),
            pl.BlockSpec((1, 1, blk), lambda j: (j, 0, 0)),
        ],
        out_specs=pl.BlockSpec((1, 1), lambda j: (0, 0)),
        out_shape=jax.ShapeDtypeStruct((1, 1), jnp.float32),
        scratch_shapes=[
            pltpu.VMEM((4, blk), jnp.float32),
            pltpu.VMEM((b, d + 2), jnp.float32),
        ],
        compiler_params=pltpu.CompilerParams(
            dimension_semantics=("arbitrary",),
        ),
    )(embeddings, lab_row, lab_col, memory_emb, mlab)
    return out[0, 0]


# speedup vs baseline: 2.6698x; 1.0347x over previous
import functools

import jax
import jax.numpy as jnp
from jax.experimental import pallas as pl
from jax.experimental.pallas import tpu as pltpu


def _accum_block(first, eaug, r, lab_col, rl_row, acc_ref):
    blk = r.shape[0]
    b = eaug.shape[0]
    m2 = jnp.sum(r * r, axis=1, keepdims=True)
    raug = jnp.concatenate(
        [r, m2, jnp.ones((blk, 1), jnp.float32)], axis=1)
    d2 = jax.lax.dot_general(
        eaug, raug, (((1,), (1,)), ((), ())),
        preferred_element_type=jnp.float32,
    )
    d2c = jnp.maximum(d2, 1e-12)
    dist = d2c * jax.lax.rsqrt(d2c)

    same = lab_col == rl_row
    if first:
        row_i = jax.lax.broadcasted_iota(jnp.int32, (b, blk), 0)
        col_i = jax.lax.broadcasted_iota(jnp.int32, (b, blk), 1)
        posm = same & (row_i != col_i)
    else:
        posm = same

    posv = jnp.where(posm, dist, 0.0)
    negv = jnp.where(same, 0.0, jnp.maximum(1.0 - dist, 0.0))
    negc = jnp.ceil(negv)
    cw = jnp.where(posm, 4096.0, negc)

    pos_s = jnp.sum(posv, axis=0, keepdims=True)
    neg_s = jnp.sum(negv, axis=0, keepdims=True)
    cw_s = jnp.sum(cw, axis=0, keepdims=True)
    pos_c = jnp.floor(cw_s * (1.0 / 4096.0))
    neg_c = cw_s - 4096.0 * pos_c

    acc_ref[...] += jnp.concatenate([pos_s, pos_c, neg_s, neg_c], axis=0)


def _body(e_ref, lab_row_ref, lab_col_ref, mem_ref, mlab_ref, out_ref,
          acc_ref, eaug_ref):
    j = pl.program_id(0)
    lab_col = lab_col_ref[...]

    @pl.when(j == 0)
    def _first():
        acc_ref[...] = jnp.zeros_like(acc_ref)
        e = e_ref[...]
        b = e.shape[0]
        q2 = jnp.sum(e * e, axis=1, keepdims=True)
        eaug = jnp.concatenate(
            [-2.0 * e, jnp.ones((b, 1), jnp.float32), q2], axis=1)
        eaug_ref[...] = eaug
        blk = mem_ref.shape[0]
        r0 = jnp.concatenate([e, mem_ref[b:blk, :]], axis=0)
        rl0 = jnp.concatenate(
            [lab_row_ref[...], mlab_ref[0][:, b:blk]], axis=1)
        _accum_block(True, eaug, r0, lab_col, rl0, acc_ref)

    @pl.when(j > 0)
    def _rest():
        _accum_block(False, eaug_ref[...], mem_ref[...], lab_col,
                     mlab_ref[0], acc_ref)

    @pl.when(j == pl.num_programs(0) - 1)
    def _final():
        s = jnp.sum(acc_ref[...], axis=1, keepdims=True)
        num = jnp.concatenate([s[0:1], s[2:3]], axis=0)
        den = jnp.maximum(jnp.concatenate([s[1:2], s[3:4]], axis=0), 1.0)
        out_ref[...] = jnp.sum(num / den, axis=0, keepdims=True)


def kernel(embeddings, labels, memory_emb, memory_labels):
    b, d = embeddings.shape
    m = memory_emb.shape[0]
    blk = 4 * b
    grid = m // blk

    lab_row = labels.reshape(1, b)
    lab_col = labels.reshape(b, 1)
    mlab = memory_labels.reshape(grid, 1, blk)

    out = pl.pallas_call(
        _body,
        grid=(grid,),
        in_specs=[
            pl.BlockSpec((b, d), lambda j: (0, 0)),
            pl.BlockSpec((1, b), lambda j: (0, 0)),
            pl.BlockSpec((b, 1), lambda j: (0, 0)),
            pl.BlockSpec((blk, d), lambda j: (j, 0)),
            pl.BlockSpec((1, 1, blk), lambda j: (j, 0, 0)),
        ],
        out_specs=pl.BlockSpec((1, 1), lambda j: (0, 0)),
        out_shape=jax.ShapeDtypeStruct((1, 1), jnp.float32),
        scratch_shapes=[
            pltpu.VMEM((4, blk), jnp.float32),
            pltpu.VMEM((b, d + 2), jnp.float32),
        ],
        compiler_params=pltpu.CompilerParams(
            dimension_semantics=("arbitrary",),
        ),
    )(embeddings, lab_row, lab_col, memory_emb, mlab)
    return out[0, 0]
